# baseline (speedup 1.0000x reference)
import functools

import jax
import jax.numpy as jnp
import numpy as np
from jax.experimental import pallas as pl
from jax.experimental.pallas import tpu as pltpu
from jax.experimental.shard_map import shard_map
from jax.sharding import Mesh, PartitionSpec as P


def _pick_bm(n: int, target: int = 400) -> int:
    best = 8
    for d in range(8, target + 1, 8):
        if n % d == 0:
            best = d
    return best


def _pass1_kernel(adj_ref, x_ref, w0_ref, b0_ref, w1_ref, b1_ref,
                  s1_ref, s0_scratch):
    @pl.when(pl.program_id(0) == 0)
    def _():
        x = jnp.maximum(x_ref[...], 0.0)
        s0_scratch[...] = (
            jnp.dot(x, w0_ref[...], preferred_element_type=jnp.float32)
            + b0_ref[...]
        )

    acc = jnp.dot(adj_ref[...], s0_scratch[...],
                  preferred_element_type=jnp.float32)
    x1 = jnp.maximum(acc, 0.0)
    s1_ref[...] = (
        jnp.dot(x1, w1_ref[...], preferred_element_type=jnp.float32)
        + b1_ref[...]
    )


def _pass2_kernel(adj_ref, s1_ref, out_ref):
    acc = jnp.dot(adj_ref[...], s1_ref[...],
                  preferred_element_type=jnp.float32)
    x2 = jnp.maximum(acc, 0.0)
    m = jnp.max(x2, axis=1, keepdims=True)
    z = x2 - m
    lse = jnp.log(jnp.sum(jnp.exp(z), axis=1, keepdims=True))
    out_ref[...] = z - lse


def _gcn_shard(adj_shard, x, w0, b0_2d, w1, b1_2d):
    m, n = adj_shard.shape
    in_size = x.shape[1]
    hidd = w0.shape[1]
    n_class = w1.shape[1]
    bm = _pick_bm(m)
    grid = (m // bm,)

    full = lambda *shape: pl.BlockSpec(shape, lambda i: (0,) * len(shape))

    s1_local = pl.pallas_call(
        _pass1_kernel,
        grid=grid,
        in_specs=[
            pl.BlockSpec((bm, n), lambda i: (i, 0)),
            full(n, in_size),
            full(in_size, hidd),
            full(1, hidd),
            full(hidd, n_class),
            full(1, n_class),
        ],
        out_specs=pl.BlockSpec((bm, n_class), lambda i: (i, 0)),
        out_shape=jax.ShapeDtypeStruct((m, n_class), jnp.float32),
        scratch_shapes=[pltpu.VMEM((n, hidd), jnp.float32)],
    )(adj_shard, x, w0, b0_2d, w1, b1_2d)

    s1_full = jax.lax.all_gather(s1_local, "x", axis=0, tiled=True)

    out_local = pl.pallas_call(
        _pass2_kernel,
        grid=grid,
        in_specs=[
            pl.BlockSpec((bm, n), lambda i: (i, 0)),
            full(n, n_class),
        ],
        out_specs=pl.BlockSpec((bm, n_class), lambda i: (i, 0)),
        out_shape=jax.ShapeDtypeStruct((m, n_class), jnp.float32),
    )(adj_shard, s1_full)

    return out_local


@jax.jit
def kernel(input, adj, W0, b0, W1, b1):
    n = adj.shape[0]
    devs = jax.devices()
    ndev = len(devs)
    while ndev > 1 and n % ndev != 0:
        ndev -= 1
    mesh = Mesh(np.array(devs[:ndev]), ("x",))

    b0_2d = b0.reshape(1, -1)
    b1_2d = b1.reshape(1, -1)

    f = shard_map(
        _gcn_shard,
        mesh=mesh,
        in_specs=(P("x", None), P(None, None), P(None, None),
                  P(None, None), P(None, None), P(None, None)),
        out_specs=P("x", None),
        check_rep=False,
    )
    return f(adj, input, W0, b0_2d, W1, b1_2d)


# speedup vs baseline: 3.2681x; 3.2681x over previous
import jax
import jax.numpy as jnp
from jax.experimental import pallas as pl
from jax.experimental.pallas import tpu as pltpu


def _pick_bm(n: int, target: int = 400) -> int:
    best = 8
    for d in range(8, target + 1, 8):
        if n % d == 0:
            best = d
    return best


def _make_gcn_kernel(bm: int):
    def _gcn_kernel(adj_ref, x_ref, w0_ref, b0_ref, w1_ref, b1_ref,
                    out_ref, s0_scratch, s1_scratch):
        ph = pl.program_id(0)
        i = pl.program_id(1)

        @pl.when((ph == 0) & (i == 0))
        def _():
            x = jnp.maximum(x_ref[...], 0.0)
            s0_scratch[...] = (
                jnp.dot(x, w0_ref[...], preferred_element_type=jnp.float32)
                + b0_ref[...]
            )

        @pl.when(ph == 0)
        def _():
            acc = jnp.dot(adj_ref[...], s0_scratch[...],
                          preferred_element_type=jnp.float32)
            x1 = jnp.maximum(acc, 0.0)
            s1_blk = (
                jnp.dot(x1, w1_ref[...], preferred_element_type=jnp.float32)
                + b1_ref[...]
            )
            s1_scratch[pl.ds(i * bm, bm), :] = s1_blk

        @pl.when(ph == 1)
        def _():
            acc = jnp.dot(adj_ref[...], s1_scratch[...],
                          preferred_element_type=jnp.float32)
            x2 = jnp.maximum(acc, 0.0)
            m = jnp.max(x2, axis=1, keepdims=True)
            z = x2 - m
            lse = jnp.log(jnp.sum(jnp.exp(z), axis=1, keepdims=True))
            out_ref[...] = z - lse

    return _gcn_kernel


@jax.jit
def kernel(input, adj, W0, b0, W1, b1):
    n, in_size = input.shape
    hidd = W0.shape[1]
    n_class = W1.shape[1]
    bm = _pick_bm(n)
    grid = (2, n // bm)

    b0_2d = b0.reshape(1, hidd)
    b1_2d = b1.reshape(1, n_class)

    full = lambda *shape: pl.BlockSpec(shape, lambda ph, i: (0,) * len(shape))

    out = pl.pallas_call(
        _make_gcn_kernel(bm),
        grid=grid,
        in_specs=[
            pl.BlockSpec((bm, n), lambda ph, i: (i, 0)),
            full(n, in_size),
            full(in_size, hidd),
            full(1, hidd),
            full(hidd, n_class),
            full(1, n_class),
        ],
        out_specs=pl.BlockSpec((bm, n_class), lambda ph, i: (ph * i, 0)),
        out_shape=jax.ShapeDtypeStruct((n, n_class), jnp.float32),
        scratch_shapes=[
            pltpu.VMEM((n, hidd), jnp.float32),
            pltpu.VMEM((n, n_class), jnp.float32),
        ],
    )(adj, input, W0, b0_2d, W1, b1_2d)

    return out
